# baseline (speedup 1.0000x reference)
import jax
import jax.numpy as jnp
from jax.experimental import pallas as pl

_BACK = 1024
_FORW = 7168
_BR = 512


def _body(x_ref, base_ref, o_ref):
    o_ref[...] = x_ref[...] - base_ref[...]


def kernel(x):
    B, N, D = x.shape
    out_rows = _FORW - _BACK
    base = x[:, 0:1, :]
    grid = (B, out_rows // _BR)
    return pl.pallas_call(
        _body,
        grid=grid,
        in_specs=[
            pl.BlockSpec((1, _BR, D), lambda b, i: (b, (_BACK // _BR) + i, 0)),
            pl.BlockSpec((1, 1, D), lambda b, i: (b, 0, 0)),
        ],
        out_specs=pl.BlockSpec((1, _BR, D), lambda b, i: (b, i, 0)),
        out_shape=jax.ShapeDtypeStruct((B, out_rows, D), x.dtype),
    )(x, base)


# speedup vs baseline: 1.0939x; 1.0939x over previous
import jax
import jax.numpy as jnp
from jax.experimental import pallas as pl

_BACK = 1024
_FORW = 7168
_BR = 1024


def _body(x_ref, base_ref, o_ref):
    o_ref[...] = x_ref[...] - base_ref[...]


def kernel(x):
    B, N, D = x.shape
    out_rows = _FORW - _BACK
    base = x[:, 0:1, :]
    grid = (B, out_rows // _BR)
    return pl.pallas_call(
        _body,
        grid=grid,
        in_specs=[
            pl.BlockSpec((1, _BR, D), lambda b, i: (b, (_BACK // _BR) + i, 0)),
            pl.BlockSpec((1, 1, D), lambda b, i: (b, 0, 0)),
        ],
        out_specs=pl.BlockSpec((1, _BR, D), lambda b, i: (b, i, 0)),
        out_shape=jax.ShapeDtypeStruct((B, out_rows, D), x.dtype),
    )(x, base)
